# baseline (speedup 1.0000x reference)
import functools

import jax
import jax.numpy as jnp
from jax import lax
from jax.experimental import pallas as pl
from jax.experimental.pallas import tpu as pltpu
from jax.experimental.pallas import tpu_sc as plsc

_B, _N, _C, _M = 8, 16384, 16, 128
_NSP = 512
_D = 3 + _C
_LANES = 16
_CHUNKS = _N // _LANES
_NW = 32
_BOXES_PER_W = (_B * _M) // _NW
_ZROW = _B * _N
_TROWS = _B * _N + 8
_GCH = _NSP // 128
_DP = 128


def _tc_masks(pts_t, boxp):
    nbx = 32

    def mask_kernel(boxp_ref, pts_ref, out_ref):
        x = pts_ref[0, 0, :].reshape(128, 128)
        y = pts_ref[0, 1, :].reshape(128, 128)
        z = pts_ref[0, 2, :].reshape(128, 128)
        col = lax.broadcasted_iota(jnp.int32, (128, 8), 0)
        grp = lax.broadcasted_iota(jnp.int32, (128, 8), 1)
        packm = jnp.where(col // 16 == grp,
                          (2.0 ** (col % 16).astype(jnp.float32)), 0.0)
        for u in range(nbx):
            pv = boxp_ref[u, 0]
            sx = x - pv[0]
            sy = y - pv[1]
            lx = sx * pv[6] - sy * pv[7]
            ly = sx * pv[7] + sy * pv[6]
            m = ((jnp.abs(z - pv[2]) <= pv[5])
                 & (lx > -pv[3]) & (lx < pv[3])
                 & (ly > -pv[4]) & (ly < pv[4]))
            w = jnp.dot(m.astype(jnp.float32), packm,
                        preferred_element_type=jnp.float32)
            out_ref[u] = w.astype(jnp.int32)

    return pl.pallas_call(
        mask_kernel,
        grid=(_B * _M // nbx,),
        in_specs=[
            pl.BlockSpec((nbx, 1, _LANES), lambda i: (i, 0, 0)),
            pl.BlockSpec((1, 3, _N), lambda i: (i // (_M // nbx), 0, 0)),
        ],
        out_specs=pl.BlockSpec((nbx, 128, 8), lambda i: (i, 0, 0)),
        out_shape=jax.ShapeDtypeStruct((_B * _M, 128, 8), jnp.int32),
    )(boxp.reshape(_B * _M, 1, _LANES), pts_t)


def _sc_pool(masks, table):
    mesh = plsc.VectorSubcoreMesh(core_axis_name="c", subcore_axis_name="s")

    @functools.partial(
        pl.kernel,
        out_type=[
            jax.ShapeDtypeStruct((_B * _M * _NSP, _DP), jnp.float32),
            jax.ShapeDtypeStruct((_B * _M,), jnp.int32),
        ],
        mesh=mesh,
        compiler_params=pltpu.CompilerParams(needs_layout_passes=False,
                                             use_tc_tiling_on_sc=False),
        scratch_types=[
            pltpu.VMEM((_BOXES_PER_W * _CHUNKS,), jnp.int32),
            pltpu.VMEM((_NSP + 2 * _LANES,), jnp.int32),
            pltpu.VMEM((2, _GCH, 128), jnp.int32),
            pltpu.VMEM((_NSP, _DP), jnp.float32),
            pltpu.VMEM((_BOXES_PER_W,), jnp.int32),
            pltpu.SemaphoreType.DMA,
        ],
    )
    def body(masks_hbm, table_hbm, out_hbm, flags_hbm,
             mask_v, idxbuf, g_v, rows_v, flags_v, sem):
        wid = lax.axis_index("s") * 2 + lax.axis_index("c")
        base_box = wid * _BOXES_PER_W
        b = base_box // _M

        pltpu.sync_copy(
            masks_hbm.at[pl.ds(base_box * _CHUNKS, _BOXES_PER_W * _CHUNKS)],
            mask_v)

        bn = b * _N
        tab2 = table_hbm
        rows2 = rows_v
        iota16 = jnp.arange(_LANES, dtype=jnp.int32)

        def box_body(i, _):
            boxw = i * _CHUNKS

            idxbuf[pl.ds(0, _LANES)] = jnp.zeros((_LANES,), jnp.int32)

            def scan_body(gi, cnt):
                c0 = gi * _LANES
                wv = mask_v[pl.ds(boxw + c0, _LANES)]
                m16 = wv != 0
                anyp = plsc.all_reduce_population_count(m16)[0]

                def do_extract(c):
                    for u in range(_LANES):
                        w = wv[u]

                        def app(c2, u=u, w=w):
                            bits = (jnp.broadcast_to(w, (_LANES,))
                                    >> iota16) & 1
                            m = bits != 0
                            idxv = iota16 + (c0 + u) * _LANES
                            plsc.store_compressed(
                                idxbuf.at[pl.ds(jnp.minimum(c2, _NSP),
                                                _LANES)],
                                idxv, mask=m)
                            return c2 + plsc.all_reduce_population_count(m)[0]

                        c = lax.cond(w != 0, app, lambda c2: c2, c)
                    return c

                return lax.cond(anyp > 0, do_extract, lambda c: c, cnt)

            cnt = lax.fori_loop(0, _CHUNKS // _LANES, scan_body, jnp.int32(0))

            empty = cnt == jnp.int32(0)
            lane0 = iota16 == 0
            plsc.store_scatter(flags_v, [jnp.broadcast_to(i, (_LANES,))],
                               jnp.broadcast_to(empty.astype(jnp.int32),
                                                (_LANES,)),
                               mask=lane0)
            kc = jnp.maximum(jnp.minimum(cnt, _NSP), 1)
            emptyv = jnp.broadcast_to(empty, (_LANES,))
            zrv = jnp.full((_LANES,), _ZROW, jnp.int32)
            bnv = jnp.broadcast_to(bn, (_LANES,))

            ib = i % 2
            for cj in range(_NSP // _LANES):
                jv = iota16 + cj * _LANES
                jm = jv % kc
                vals = plsc.load_gather(idxbuf, [jm])
                gidx = jnp.where(emptyv, zrv, vals + bnv)
                g_v[ib, cj * _LANES // 128,
                    pl.ds((cj * _LANES) % 128, _LANES)] = gidx

            @pl.when(i > 0)
            def _():
                for q in range(_GCH):
                    pltpu.make_async_copy(
                        tab2.at[pl.ds(0, 128)],
                        rows2.at[pl.ds(q * 128, 128)], sem).wait()
                pltpu.sync_copy(
                    rows_v.at[:, pl.ds(0, 24)],
                    out_hbm.at[pl.ds((base_box + i - 1) * _NSP, _NSP),
                               pl.ds(0, 24)])

            for q in range(_GCH):
                pltpu.async_copy(tab2.at[g_v.at[ib, q]],
                                 rows2.at[pl.ds(q * 128, 128)], sem)
            return 0

        lax.fori_loop(0, _BOXES_PER_W, box_body, 0)
        for q in range(_GCH):
            pltpu.make_async_copy(tab2.at[pl.ds(0, 128)],
                                  rows2.at[pl.ds(q * 128, 128)], sem).wait()
        pltpu.sync_copy(
            rows_v.at[:, pl.ds(0, 24)],
            out_hbm.at[pl.ds((base_box + _BOXES_PER_W - 1) * _NSP, _NSP),
                       pl.ds(0, 24)])
        pltpu.sync_copy(flags_v, flags_hbm.at[pl.ds(base_box, _BOXES_PER_W)])

    return body(masks, table)


def kernel(points, point_features, boxes3d, num_sampled_points):
    del num_sampled_points
    pts_t = jnp.transpose(points, (0, 2, 1))
    table = jnp.concatenate([points, point_features], axis=-1)
    table = table.reshape(_B * _N, _D)
    table = jnp.pad(table, ((0, 8), (0, _DP - _D)))

    cx = boxes3d[..., 0]
    cy = boxes3d[..., 1]
    czc = boxes3d[..., 2] + 0.5 * boxes3d[..., 5]
    hx = 0.5 * boxes3d[..., 3]
    hy = 0.5 * boxes3d[..., 4]
    hz = 0.5 * boxes3d[..., 5]
    ca = jnp.cos(-boxes3d[..., 6])
    sa = jnp.sin(-boxes3d[..., 6])
    zpad = jnp.zeros_like(cx)
    boxp = jnp.stack([cx, cy, czc, hx, hy, hz, ca, sa]
                     + [zpad] * 8, axis=-1)
    boxp = boxp.reshape(_B * _M, _LANES)

    masks = _tc_masks(pts_t, boxp)
    pooled_flat, flags = _sc_pool(masks.reshape(-1), table)
    pooled = pooled_flat.reshape(_B, _M, _NSP, _DP)[..., :_D]
    return pooled, flags.reshape(_B, _M)


# speedup vs baseline: 1.1324x; 1.1324x over previous
import functools

import jax
import jax.numpy as jnp
from jax import lax
from jax.experimental import pallas as pl
from jax.experimental.pallas import tpu as pltpu
from jax.experimental.pallas import tpu_sc as plsc

_B, _N, _C, _M = 8, 16384, 16, 128
_NSP = 512
_D = 3 + _C
_LANES = 16
_CHUNKS = _N // _LANES
_NW = 32
_BOXES_PER_W = (_B * _M) // _NW
_ZROW = _B * _N
_TROWS = _B * _N + 8
_GCH = _NSP // 128
_GRP = 32
_ROWW = _NSP * _D
_DP = 128


def _sc_pool(pts_t, table, boxp):
    mesh = plsc.VectorSubcoreMesh(core_axis_name="c", subcore_axis_name="s")

    @functools.partial(
        pl.kernel,
        out_type=[
            jax.ShapeDtypeStruct((_B * _M * _NSP, _DP), jnp.float32),
            jax.ShapeDtypeStruct((_B * _M,), jnp.int32),
        ],
        mesh=mesh,
        compiler_params=pltpu.CompilerParams(needs_layout_passes=False,
                                             use_tc_tiling_on_sc=False),
        scratch_types=[
            pltpu.VMEM((3 * _N,), jnp.float32),
            pltpu.VMEM((_BOXES_PER_W * _LANES,), jnp.float32),
            pltpu.VMEM((_NSP + 2 * _LANES,), jnp.int32),
            pltpu.VMEM((2, _GCH, 128), jnp.int32),
            pltpu.VMEM((_NSP, _DP), jnp.float32),
            pltpu.VMEM((_BOXES_PER_W,), jnp.int32),
            pltpu.SemaphoreType.DMA,
            pltpu.SemaphoreType.DMA,
        ],
    )
    def body(pts_hbm, table_hbm, boxp_hbm, out_hbm, flags_hbm,
             pts_v, boxp_v, idxbuf, g_v, rows_v, flags_v, sem, wsem):
        wid = lax.axis_index("s") * 2 + lax.axis_index("c")
        base_box = wid * _BOXES_PER_W
        b = base_box // _M

        pltpu.sync_copy(pts_hbm.at[pl.ds(b * 3 * _N, 3 * _N)], pts_v)
        pltpu.sync_copy(
            boxp_hbm.at[pl.ds(base_box * _LANES, _BOXES_PER_W * _LANES)],
            boxp_v)

        bn = b * _N
        tab2 = table_hbm
        rows2 = rows_v

        def box_body(i, _):
            pv = boxp_v[pl.ds(i * _LANES, _LANES)]
            cxv = jnp.broadcast_to(pv[0], (_LANES,))
            cyv = jnp.broadcast_to(pv[1], (_LANES,))
            czv = jnp.broadcast_to(pv[2], (_LANES,))
            hxv = jnp.broadcast_to(pv[3], (_LANES,))
            hyv = jnp.broadcast_to(pv[4], (_LANES,))
            hzv = jnp.broadcast_to(pv[5], (_LANES,))
            cav = jnp.broadcast_to(pv[6], (_LANES,))
            sav = jnp.broadcast_to(pv[7], (_LANES,))
            nhxv = -hxv
            nhyv = -hyv

            idxbuf[pl.ds(0, _LANES)] = jnp.zeros((_LANES,), jnp.int32)

            def scan_body(gi, cnt):
                goff = gi * (_LANES * _GRP)
                ms = []
                for u in range(_GRP):
                    off = goff + u * _LANES
                    x = pts_v[pl.ds(off, _LANES)]
                    y = pts_v[pl.ds(_N + off, _LANES)]
                    z = pts_v[pl.ds(2 * _N + off, _LANES)]
                    sx = x - cxv
                    sy = y - cyv
                    lx = sx * cav - sy * sav
                    ly = sx * sav + sy * cav
                    ms.append((jnp.abs(z - czv) <= hzv)
                              & (lx > nhxv) & (lx < hxv)
                              & (ly > nhyv) & (ly < hyv))

                mo = ms[0]
                for u in range(1, _GRP):
                    mo = mo | ms[u]
                anyp = plsc.all_reduce_population_count(mo)[0]

                def do_append(c):
                    nps = [plsc.all_reduce_population_count(ms[u])[0]
                           for u in range(_GRP)]
                    offs = []
                    for u in range(_GRP):
                        offs.append(c)
                        c = c + nps[u]
                    for u in range(_GRP):
                        idxv = (jnp.arange(_LANES, dtype=jnp.int32)
                                + (goff + u * _LANES))
                        plsc.store_compressed(
                            idxbuf.at[pl.ds(jnp.minimum(offs[u], _NSP),
                                            _LANES)],
                            idxv, mask=ms[u])
                    return c

                return lax.cond(anyp > 0, do_append, lambda c: c, cnt)

            cnt = lax.fori_loop(0, _CHUNKS // _GRP, scan_body, jnp.int32(0))

            empty = cnt == jnp.int32(0)
            lane0 = jnp.arange(_LANES, dtype=jnp.int32) == 0
            plsc.store_scatter(flags_v, [jnp.broadcast_to(i, (_LANES,))],
                               jnp.broadcast_to(empty.astype(jnp.int32),
                                                (_LANES,)),
                               mask=lane0)
            kc = jnp.maximum(jnp.minimum(cnt, _NSP), 1)
            emptyv = jnp.broadcast_to(empty, (_LANES,))
            zrv = jnp.full((_LANES,), _ZROW, jnp.int32)
            bnv = jnp.broadcast_to(bn, (_LANES,))

            ib = i % 2
            tstep = jnp.broadcast_to(jnp.int32(_LANES) % kc, (_LANES,))
            kcv = jnp.broadcast_to(kc, (_LANES,))
            jm = jnp.arange(_LANES, dtype=jnp.int32) % kcv
            for cj in range(_NSP // _LANES):
                vals = plsc.load_gather(idxbuf, [jm])
                gidx = jnp.where(emptyv, zrv, vals + bnv)
                g_v[ib, cj * _LANES // 128,
                    pl.ds((cj * _LANES) % 128, _LANES)] = gidx
                jmn = jm + tstep
                jm = jnp.where(jmn >= kcv, jmn - kcv, jmn)

            @pl.when(i > 0)
            def _():
                for q in range(_GCH):
                    pltpu.make_async_copy(
                        tab2.at[pl.ds(0, 128)],
                        rows2.at[pl.ds(q * 128, 128)], sem).wait()
                pltpu.sync_copy(
                    rows_v.at[:, pl.ds(0, 24)],
                    out_hbm.at[pl.ds((base_box + i - 1) * _NSP, _NSP),
                               pl.ds(0, 24)])

            for q in range(_GCH):
                pltpu.async_copy(tab2.at[g_v.at[ib, q]],
                                 rows2.at[pl.ds(q * 128, 128)], sem)
            return 0

        lax.fori_loop(0, _BOXES_PER_W, box_body, 0)
        for q in range(_GCH):
            pltpu.make_async_copy(tab2.at[pl.ds(0, 128)],
                                  rows2.at[pl.ds(q * 128, 128)], sem).wait()
        pltpu.sync_copy(
            rows_v.at[:, pl.ds(0, 24)],
            out_hbm.at[pl.ds((base_box + _BOXES_PER_W - 1) * _NSP, _NSP),
                       pl.ds(0, 24)])
        pltpu.sync_copy(flags_v, flags_hbm.at[pl.ds(base_box, _BOXES_PER_W)])

    return body(pts_t, table, boxp)


def kernel(points, point_features, boxes3d, num_sampled_points):
    del num_sampled_points
    pts_t = jnp.transpose(points, (0, 2, 1))
    table = jnp.concatenate([points, point_features], axis=-1)
    table = table.reshape(_B * _N, _D)
    table = jnp.pad(table, ((0, 8), (0, _DP - _D)))

    cx = boxes3d[..., 0]
    cy = boxes3d[..., 1]
    czc = boxes3d[..., 2] + 0.5 * boxes3d[..., 5]
    hx = 0.5 * boxes3d[..., 3]
    hy = 0.5 * boxes3d[..., 4]
    hz = 0.5 * boxes3d[..., 5]
    ca = jnp.cos(boxes3d[..., 6])
    sa = -jnp.sin(boxes3d[..., 6])
    zpad = jnp.zeros_like(cx)
    boxp = jnp.stack([cx, cy, czc, hx, hy, hz, ca, sa] + [zpad] * 8, axis=-1)
    boxp = boxp.reshape(_B * _M, _LANES)

    pooled_flat, flags = _sc_pool(
        pts_t.reshape(-1), table, boxp.reshape(-1))
    pooled = pooled_flat.reshape(_B, _M, _NSP, _DP)[..., :_D]
    return pooled, flags.reshape(_B, _M)


# speedup vs baseline: 1.1343x; 1.0017x over previous
import functools

import jax
import jax.numpy as jnp
from jax import lax
from jax.experimental import pallas as pl
from jax.experimental.pallas import tpu as pltpu
from jax.experimental.pallas import tpu_sc as plsc

_B, _N, _C, _M = 8, 16384, 16, 128
_NSP = 512
_D = 3 + _C
_LANES = 16
_CHUNKS = _N // _LANES
_NW = 32
_BOXES_PER_W = (_B * _M) // _NW
_ZROW = _B * _N
_TROWS = _B * _N + 8
_GCH = _NSP // 128
_GRP = 32
_ROWW = _NSP * _D
_DP = 128


def _sc_pool(pts_t, table, boxp):
    mesh = plsc.VectorSubcoreMesh(core_axis_name="c", subcore_axis_name="s")

    @functools.partial(
        pl.kernel,
        out_type=[
            jax.ShapeDtypeStruct((_B * _M * _NSP, _DP), jnp.float32),
            jax.ShapeDtypeStruct((_B * _M,), jnp.int32),
        ],
        mesh=mesh,
        compiler_params=pltpu.CompilerParams(needs_layout_passes=False,
                                             use_tc_tiling_on_sc=False),
        scratch_types=[
            pltpu.VMEM((3 * _N,), jnp.float32),
            pltpu.VMEM((_BOXES_PER_W * _LANES,), jnp.float32),
            pltpu.VMEM((_NSP + 2 * _LANES,), jnp.int32),
            pltpu.VMEM((2, _GCH, 128), jnp.int32),
            pltpu.VMEM((_NSP, _DP), jnp.float32),
            pltpu.VMEM((_BOXES_PER_W,), jnp.int32),
            pltpu.SemaphoreType.DMA,
            pltpu.SemaphoreType.DMA,
        ],
    )
    def body(pts_hbm, table_hbm, boxp_hbm, out_hbm, flags_hbm,
             pts_v, boxp_v, idxbuf, g_v, rows_v, flags_v, sem, wsem):
        wid = lax.axis_index("s") * 2 + lax.axis_index("c")
        base_box = wid * _BOXES_PER_W
        b = base_box // _M

        pltpu.sync_copy(pts_hbm.at[pl.ds(b * 3 * _N, 3 * _N)], pts_v)
        pltpu.sync_copy(
            boxp_hbm.at[pl.ds(base_box * _LANES, _BOXES_PER_W * _LANES)],
            boxp_v)

        bn = b * _N
        tab2 = table_hbm
        rows2 = rows_v

        def box_body(i, _):
            pv = boxp_v[pl.ds(i * _LANES, _LANES)]
            cxv = jnp.broadcast_to(pv[0], (_LANES,))
            cyv = jnp.broadcast_to(pv[1], (_LANES,))
            czv = jnp.broadcast_to(pv[2], (_LANES,))
            hxv = jnp.broadcast_to(pv[3], (_LANES,))
            hyv = jnp.broadcast_to(pv[4], (_LANES,))
            hzv = jnp.broadcast_to(pv[5], (_LANES,))
            cav = jnp.broadcast_to(pv[6], (_LANES,))
            sav = jnp.broadcast_to(pv[7], (_LANES,))
            nhxv = -hxv
            nhyv = -hyv

            idxbuf[pl.ds(0, _LANES)] = jnp.full((_LANES,), _ZROW - bn,
                                                jnp.int32)

            def scan_body(gi, cnt):
                goff = gi * (_LANES * _GRP)
                ms = []
                for u in range(_GRP):
                    off = goff + u * _LANES
                    x = pts_v[pl.ds(off, _LANES)]
                    y = pts_v[pl.ds(_N + off, _LANES)]
                    z = pts_v[pl.ds(2 * _N + off, _LANES)]
                    sx = x - cxv
                    sy = y - cyv
                    lx = sx * cav - sy * sav
                    ly = sx * sav + sy * cav
                    ms.append((jnp.abs(z - czv) <= hzv)
                              & (lx > nhxv) & (lx < hxv)
                              & (ly > nhyv) & (ly < hyv))

                mo = ms[0]
                for u in range(1, _GRP):
                    mo = mo | ms[u]
                anyp = plsc.all_reduce_population_count(mo)[0]

                def do_append(c):
                    nps = [plsc.all_reduce_population_count(ms[u])[0]
                           for u in range(_GRP)]
                    offs = []
                    for u in range(_GRP):
                        offs.append(c)
                        c = c + nps[u]
                    for u in range(_GRP):
                        idxv = (jnp.arange(_LANES, dtype=jnp.int32)
                                + (goff + u * _LANES))
                        plsc.store_compressed(
                            idxbuf.at[pl.ds(jnp.minimum(offs[u], _NSP),
                                            _LANES)],
                            idxv, mask=ms[u])
                    return c

                return lax.cond(anyp > 0, do_append, lambda c: c, cnt)

            cnt = lax.fori_loop(0, _CHUNKS // _GRP, scan_body, jnp.int32(0))

            empty = cnt == jnp.int32(0)
            lane0 = jnp.arange(_LANES, dtype=jnp.int32) == 0
            plsc.store_scatter(flags_v, [jnp.broadcast_to(i, (_LANES,))],
                               jnp.broadcast_to(empty.astype(jnp.int32),
                                                (_LANES,)),
                               mask=lane0)
            kc = jnp.maximum(jnp.minimum(cnt, _NSP), 1)
            bnv = jnp.broadcast_to(bn, (_LANES,))

            ib = i % 2
            tstep = jnp.broadcast_to(jnp.int32(_LANES) % kc, (_LANES,))
            kcv = jnp.broadcast_to(kc, (_LANES,))
            jm = jnp.arange(_LANES, dtype=jnp.int32) % kcv
            for cj in range(_NSP // _LANES):
                vals = plsc.load_gather(idxbuf, [jm])
                gidx = vals + bnv
                g_v[ib, cj * _LANES // 128,
                    pl.ds((cj * _LANES) % 128, _LANES)] = gidx
                jmn = jm + tstep
                jm = jnp.where(jmn >= kcv, jmn - kcv, jmn)

            @pl.when(i > 0)
            def _():
                for q in range(_GCH):
                    pltpu.make_async_copy(
                        tab2.at[pl.ds(0, 128)],
                        rows2.at[pl.ds(q * 128, 128)], sem).wait()
                pltpu.sync_copy(
                    rows_v.at[:, pl.ds(0, 24)],
                    out_hbm.at[pl.ds((base_box + i - 1) * _NSP, _NSP),
                               pl.ds(0, 24)])

            for q in range(_GCH):
                pltpu.async_copy(tab2.at[g_v.at[ib, q]],
                                 rows2.at[pl.ds(q * 128, 128)], sem)
            return 0

        lax.fori_loop(0, _BOXES_PER_W, box_body, 0)
        for q in range(_GCH):
            pltpu.make_async_copy(tab2.at[pl.ds(0, 128)],
                                  rows2.at[pl.ds(q * 128, 128)], sem).wait()
        pltpu.sync_copy(
            rows_v.at[:, pl.ds(0, 24)],
            out_hbm.at[pl.ds((base_box + _BOXES_PER_W - 1) * _NSP, _NSP),
                       pl.ds(0, 24)])
        pltpu.sync_copy(flags_v, flags_hbm.at[pl.ds(base_box, _BOXES_PER_W)])

    return body(pts_t, table, boxp)


def kernel(points, point_features, boxes3d, num_sampled_points):
    del num_sampled_points
    pts_t = jnp.transpose(points, (0, 2, 1))
    table = jnp.concatenate([points, point_features], axis=-1)
    table = table.reshape(_B * _N, _D)
    table = jnp.pad(table, ((0, 8), (0, _DP - _D)))

    cx = boxes3d[..., 0]
    cy = boxes3d[..., 1]
    czc = boxes3d[..., 2] + 0.5 * boxes3d[..., 5]
    hx = 0.5 * boxes3d[..., 3]
    hy = 0.5 * boxes3d[..., 4]
    hz = 0.5 * boxes3d[..., 5]
    ca = jnp.cos(boxes3d[..., 6])
    sa = -jnp.sin(boxes3d[..., 6])
    zpad = jnp.zeros_like(cx)
    boxp = jnp.stack([cx, cy, czc, hx, hy, hz, ca, sa] + [zpad] * 8, axis=-1)
    boxp = boxp.reshape(_B * _M, _LANES)

    pooled_flat, flags = _sc_pool(
        pts_t.reshape(-1), table, boxp.reshape(-1))
    pooled = pooled_flat.reshape(_B, _M, _NSP, _DP)[..., :_D]
    return pooled, flags.reshape(_B, _M)


# speedup vs baseline: 1.1374x; 1.0027x over previous
import functools

import jax
import jax.numpy as jnp
from jax import lax
from jax.experimental import pallas as pl
from jax.experimental.pallas import tpu as pltpu
from jax.experimental.pallas import tpu_sc as plsc

_B, _N, _C, _M = 8, 16384, 16, 128
_NSP = 512
_D = 3 + _C
_LANES = 16
_CHUNKS = _N // _LANES
_NW = 32
_BOXES_PER_W = (_B * _M) // _NW
_ZROW = _B * _N
_TROWS = _B * _N + 8
_GCH = _NSP // 128
_GRP = 32
_ROWW = _NSP * _D
_DP = 128


def _sc_pool(pts_t, table, boxp):
    mesh = plsc.VectorSubcoreMesh(core_axis_name="c", subcore_axis_name="s")

    @functools.partial(
        pl.kernel,
        out_type=[
            jax.ShapeDtypeStruct((_B * _M * _NSP, _DP), jnp.float32),
            jax.ShapeDtypeStruct((_B * _M,), jnp.int32),
        ],
        mesh=mesh,
        compiler_params=pltpu.CompilerParams(needs_layout_passes=False,
                                             use_tc_tiling_on_sc=False),
        scratch_types=[
            pltpu.VMEM((3 * _N,), jnp.float32),
            pltpu.VMEM((_BOXES_PER_W * _LANES,), jnp.float32),
            pltpu.VMEM((_NSP + 2 * _LANES,), jnp.int32),
            pltpu.VMEM((2, _GCH, 128), jnp.int32),
            pltpu.VMEM((_NSP, _DP), jnp.float32),
            pltpu.VMEM((_BOXES_PER_W,), jnp.int32),
            pltpu.SemaphoreType.DMA,
            pltpu.SemaphoreType.DMA,
        ],
    )
    def body(pts_hbm, table_hbm, boxp_hbm, out_hbm, flags_hbm,
             pts_v, boxp_v, idxbuf, g_v, rows_v, flags_v, sem, wsem):
        wid = lax.axis_index("s") * 2 + lax.axis_index("c")
        base_box = wid * _BOXES_PER_W
        b = base_box // _M

        pltpu.sync_copy(pts_hbm.at[pl.ds(b * 3 * _N, 3 * _N)], pts_v)
        pltpu.sync_copy(
            boxp_hbm.at[pl.ds(base_box * _LANES, _BOXES_PER_W * _LANES)],
            boxp_v)

        bn = b * _N
        tab2 = table_hbm
        rows2 = rows_v

        def box_body(i, _):
            pv = boxp_v[pl.ds(i * _LANES, _LANES)]
            cxv = jnp.broadcast_to(pv[0], (_LANES,))
            cyv = jnp.broadcast_to(pv[1], (_LANES,))
            czv = jnp.broadcast_to(pv[2], (_LANES,))
            hxv = jnp.broadcast_to(pv[3], (_LANES,))
            hyv = jnp.broadcast_to(pv[4], (_LANES,))
            hzv = jnp.broadcast_to(pv[5], (_LANES,))
            cav = jnp.broadcast_to(pv[6], (_LANES,))
            sav = jnp.broadcast_to(pv[7], (_LANES,))

            idxbuf[pl.ds(0, _LANES)] = jnp.full((_LANES,), _ZROW - bn,
                                                jnp.int32)

            def scan_body(gi, cnt):
                goff = gi * (_LANES * _GRP)
                ms = []
                for u in range(_GRP):
                    off = goff + u * _LANES
                    x = pts_v[pl.ds(off, _LANES)]
                    y = pts_v[pl.ds(_N + off, _LANES)]
                    z = pts_v[pl.ds(2 * _N + off, _LANES)]
                    sx = x - cxv
                    sy = y - cyv
                    lx = sx * cav - sy * sav
                    ly = sx * sav + sy * cav
                    ms.append((jnp.abs(z - czv) <= hzv)
                              & (jnp.abs(lx) < hxv)
                              & (jnp.abs(ly) < hyv))

                mo = ms[0]
                for u in range(1, _GRP):
                    mo = mo | ms[u]
                anyp = plsc.all_reduce_population_count(mo)[0]

                def do_append(c):
                    nps = [plsc.all_reduce_population_count(ms[u])[0]
                           for u in range(_GRP)]
                    offs = []
                    for u in range(_GRP):
                        offs.append(c)
                        c = c + nps[u]
                    for u in range(_GRP):
                        idxv = (jnp.arange(_LANES, dtype=jnp.int32)
                                + (goff + u * _LANES))
                        plsc.store_compressed(
                            idxbuf.at[pl.ds(jnp.minimum(offs[u], _NSP),
                                            _LANES)],
                            idxv, mask=ms[u])
                    return c

                return lax.cond(anyp > 0, do_append, lambda c: c, cnt)

            cnt = lax.fori_loop(0, _CHUNKS // _GRP, scan_body, jnp.int32(0))

            empty = cnt == jnp.int32(0)
            lane0 = jnp.arange(_LANES, dtype=jnp.int32) == 0
            plsc.store_scatter(flags_v, [jnp.broadcast_to(i, (_LANES,))],
                               jnp.broadcast_to(empty.astype(jnp.int32),
                                                (_LANES,)),
                               mask=lane0)
            kc = jnp.maximum(jnp.minimum(cnt, _NSP), 1)
            bnv = jnp.broadcast_to(bn, (_LANES,))

            ib = i % 2
            tstep = jnp.broadcast_to(jnp.int32(_LANES) % kc, (_LANES,))
            kcv = jnp.broadcast_to(kc, (_LANES,))
            jm = jnp.arange(_LANES, dtype=jnp.int32) % kcv
            for cj in range(_NSP // _LANES):
                vals = plsc.load_gather(idxbuf, [jm])
                gidx = vals + bnv
                g_v[ib, cj * _LANES // 128,
                    pl.ds((cj * _LANES) % 128, _LANES)] = gidx
                jmn = jm + tstep
                jm = jnp.where(jmn >= kcv, jmn - kcv, jmn)

            @pl.when(i > 0)
            def _():
                for q in range(_GCH):
                    pltpu.make_async_copy(
                        tab2.at[pl.ds(0, 128)],
                        rows2.at[pl.ds(q * 128, 128)], sem).wait()
                pltpu.sync_copy(
                    rows_v.at[:, pl.ds(0, 24)],
                    out_hbm.at[pl.ds((base_box + i - 1) * _NSP, _NSP),
                               pl.ds(0, 24)])

            for q in range(_GCH):
                pltpu.async_copy(tab2.at[g_v.at[ib, q]],
                                 rows2.at[pl.ds(q * 128, 128)], sem)
            return 0

        lax.fori_loop(0, _BOXES_PER_W, box_body, 0)
        for q in range(_GCH):
            pltpu.make_async_copy(tab2.at[pl.ds(0, 128)],
                                  rows2.at[pl.ds(q * 128, 128)], sem).wait()
        pltpu.sync_copy(
            rows_v.at[:, pl.ds(0, 24)],
            out_hbm.at[pl.ds((base_box + _BOXES_PER_W - 1) * _NSP, _NSP),
                       pl.ds(0, 24)])
        pltpu.sync_copy(flags_v, flags_hbm.at[pl.ds(base_box, _BOXES_PER_W)])

    return body(pts_t, table, boxp)


def kernel(points, point_features, boxes3d, num_sampled_points):
    del num_sampled_points
    pts_t = jnp.transpose(points, (0, 2, 1))
    table = jnp.concatenate([points, point_features], axis=-1)
    table = table.reshape(_B * _N, _D)
    table = jnp.pad(table, ((0, 8), (0, _DP - _D)))

    cx = boxes3d[..., 0]
    cy = boxes3d[..., 1]
    czc = boxes3d[..., 2] + 0.5 * boxes3d[..., 5]
    hx = 0.5 * boxes3d[..., 3]
    hy = 0.5 * boxes3d[..., 4]
    hz = 0.5 * boxes3d[..., 5]
    ca = jnp.cos(boxes3d[..., 6])
    sa = -jnp.sin(boxes3d[..., 6])
    zpad = jnp.zeros_like(cx)
    boxp = jnp.stack([cx, cy, czc, hx, hy, hz, ca, sa] + [zpad] * 8, axis=-1)
    boxp = boxp.reshape(_B * _M, _LANES)

    pooled_flat, flags = _sc_pool(
        pts_t.reshape(-1), table, boxp.reshape(-1))
    pooled = pooled_flat.reshape(_B, _M, _NSP, _DP)[..., :_D]
    return pooled, flags.reshape(_B, _M)
